# baseline (speedup 1.0000x reference)
import jax
import jax.numpy as jnp
from jax import lax
from jax.experimental import pallas as pl
from jax.experimental.pallas import tpu as pltpu
from jax.experimental.pallas import tpu_sc as plsc

NC = 2
NS = 16
NW = NC * NS
B = 16384
V = 100000
D = 32
QB = 4096
NQ = B // QB
L = 16


def _gather_pass(row, idxb, outb):
  @plsc.parallel_loop(0, QB, L, unroll=8)
  def _(off):
    idxv = idxb[pl.ds(off, L)]
    outb[pl.ds(off, L)] = plsc.load_gather(row, [idxv])


def _embed_t(v_t, j_t, v_idx, j_idx, ot, row,
             idxb0, idxb1, outb0, outb1,
             semr, semi0, semi1, semo0, semo1):
  f = lax.axis_index("s") * NC + lax.axis_index("c")
  idxb = (idxb0, idxb1)
  outb = (outb0, outb1)
  semi = (semi0, semi1)
  semo = (semo0, semo1)

  row_cp = pltpu.async_copy(v_t.at[f], row, semr)
  idx_cp = [None, None]
  out_cp = [None, None]
  idx_cp[0] = pltpu.async_copy(v_idx.at[pl.ds(0, QB)], idxb[0], semi[0])

  tables = ((v_idx, 0), (j_idx, D))
  for t, (idx_hbm, obase) in enumerate(tables):
    for q in range(NQ):
      p = q % 2
      np_ = (q + 1) % 2
      idx_cp[p].wait()
      if q + 1 < NQ:
        idx_cp[np_] = pltpu.async_copy(
            idx_hbm.at[pl.ds((q + 1) * QB, QB)], idxb[np_], semi[np_])
      elif t == 0:
        idx_cp[np_] = pltpu.async_copy(
            j_idx.at[pl.ds(0, QB)], idxb[np_], semi[np_])
      if q == 0:
        row_cp.wait()
      if out_cp[p] is not None:
        out_cp[p].wait()
      _gather_pass(row, idxb[p], outb[p])
      out_cp[p] = pltpu.async_copy(
          outb[p], ot.at[obase + f, pl.ds(q * QB, QB)], semo[p])
    if t == 0:
      row_cp = pltpu.async_copy(j_t.at[f], row, semr)
  out_cp[0].wait()
  out_cp[1].wait()


@jax.jit
def _run(v_t, j_t, v_idx, j_idx):
  mesh = plsc.VectorSubcoreMesh(core_axis_name="c", subcore_axis_name="s")
  ot = pl.kernel(
      _embed_t,
      out_type=jax.ShapeDtypeStruct((2 * D, B), jnp.float32),
      mesh=mesh,
      compiler_params=pltpu.CompilerParams(needs_layout_passes=False),
      scratch_types=[
          pltpu.VMEM((V,), jnp.float32),
          pltpu.VMEM((QB,), jnp.int32),
          pltpu.VMEM((QB,), jnp.int32),
          pltpu.VMEM((QB,), jnp.float32),
          pltpu.VMEM((QB,), jnp.float32),
          pltpu.SemaphoreType.DMA,
          pltpu.SemaphoreType.DMA,
          pltpu.SemaphoreType.DMA,
          pltpu.SemaphoreType.DMA,
          pltpu.SemaphoreType.DMA,
      ],
  )(v_t, j_t, v_idx, j_idx)
  return ot.T


def kernel(v_idx, j_idx, v_table, j_table):
  return _run(v_table.T, j_table.T,
              v_idx.astype(jnp.int32), j_idx.astype(jnp.int32))
